# baseline (speedup 1.0000x reference)
import functools

import jax
import jax.numpy as jnp
from jax.experimental import pallas as pl
from jax.experimental.pallas import tpu as pltpu

_N_BINS = 15
_BN = 16384


def _accum(conf, correct, bounds, out_ref):
    g = (conf > bounds).astype(jnp.float32)
    z = jnp.zeros((1, 128), jnp.float32)
    out_ref[0] = jnp.concatenate(
        [
            jnp.sum(g, axis=0, keepdims=True),
            jnp.sum(g * conf, axis=0, keepdims=True),
            jnp.sum(g * correct, axis=0, keepdims=True),
            z, z, z, z, z,
        ],
        axis=0,
    )


def _pass1_body(bounds_ref, yp_ref, yt_ref, out_ref, *, n_rows: int):
    i = pl.program_id(0)
    nb = pl.num_programs(0)

    logits = yp_ref[...]
    m = jnp.max(logits, axis=1, keepdims=True)
    se = jnp.sum(jnp.exp(logits - m), axis=1, keepdims=True)
    conf = 1.0 / se
    pred = jnp.argmax(logits, axis=1)[:, None]
    yt = yt_ref[...][:, None]
    correct = (pred == yt).astype(jnp.float32)
    bounds = bounds_ref[...]

    @pl.when(i < nb - 1)
    def _full():
        _accum(conf, correct, bounds, out_ref)

    @pl.when(i == nb - 1)
    def _tail():
        valid = n_rows - (nb - 1) * _BN
        rloc = jax.lax.broadcasted_iota(jnp.int32, (_BN, 1), 0)
        _accum(jnp.where(rloc < valid, conf, 2.0), correct, bounds, out_ref)


def _combine_body(parts_ref, out_ref, *, inv_n: float):
    s = jnp.sum(parts_ref[...], axis=0)
    cnt = s[0:1, 0:_N_BINS] - s[0:1, 1:_N_BINS + 1]
    dcf = s[1:2, 0:_N_BINS] - s[1:2, 1:_N_BINS + 1]
    dac = s[2:3, 0:_N_BINS] - s[2:3, 1:_N_BINS + 1]
    denom = jnp.maximum(cnt, 1.0)
    term = jnp.where(cnt > 0, jnp.abs(dcf - dac) / denom * (cnt * inv_n), 0.0)
    ece = jnp.sum(term, axis=1, keepdims=True)
    out_ref[...] = jnp.broadcast_to(ece, (1, 128))


def kernel(y_pred, y_true):
    n, c = y_pred.shape
    nb = (n + _BN - 1) // _BN

    bb = jnp.linspace(0.0, 1.0, _N_BINS + 1).astype(jnp.float32)
    bounds = jnp.concatenate(
        [bb, jnp.full((128 - (_N_BINS + 1),), 2.0, jnp.float32)]
    ).reshape(1, 128)
    yt = y_true.astype(jnp.int32)

    parts = pl.pallas_call(
        functools.partial(_pass1_body, n_rows=n),
        grid=(nb,),
        in_specs=[
            pl.BlockSpec((1, 128), lambda i: (0, 0)),
            pl.BlockSpec((_BN, c), lambda i: (i, 0)),
            pl.BlockSpec((_BN,), lambda i: (i,)),
        ],
        out_specs=pl.BlockSpec((1, 8, 128), lambda i: (i, 0, 0)),
        out_shape=jax.ShapeDtypeStruct((nb, 8, 128), jnp.float32),
        compiler_params=pltpu.CompilerParams(
            dimension_semantics=("parallel",),
        ),
    )(bounds, y_pred, yt)

    out = pl.pallas_call(
        functools.partial(_combine_body, inv_n=1.0 / n),
        grid=(1,),
        in_specs=[pl.BlockSpec((nb, 8, 128), lambda i: (0, 0, 0))],
        out_specs=pl.BlockSpec((1, 128), lambda i: (0, 0)),
        out_shape=jax.ShapeDtypeStruct((1, 128), jnp.float32),
    )(parts)
    return out[0, 0:1]


# speedup vs baseline: 1.7189x; 1.7189x over previous
import functools

import jax
import jax.numpy as jnp
import numpy as np
from jax.experimental import pallas as pl
from jax.experimental.pallas import tpu as pltpu

_N_BINS = 15
_BN = 8192
_R = _BN // 128
_BOUNDS = tuple(float(x) for x in np.linspace(0.0, 1.0, _N_BINS + 1, dtype=np.float32))


def _binning(cf, cr, out_ref):
    s_cnt, s_conf, s_acc = [], [], []
    one = jnp.ones_like(cf)
    zero = jnp.zeros_like(cf)
    for b in _BOUNDS:
        g = cf > b
        s_cnt.append(jnp.sum(jnp.where(g, one, zero), axis=0, keepdims=True))
        s_conf.append(jnp.sum(jnp.where(g, cf, zero), axis=0, keepdims=True))
        s_acc.append(jnp.sum(jnp.where(g, cr, zero), axis=0, keepdims=True))
    out_ref[0] = jnp.concatenate(s_cnt + s_conf + s_acc, axis=0)


def _pass1_body(yp_ref, yt_ref, out_ref, *, n_rows: int, c: int):
    i = pl.program_id(0)
    nb = pl.num_programs(0)

    logits = yp_ref[...]
    lt = jnp.transpose(logits, (1, 0))
    m = jnp.max(lt, axis=0, keepdims=True)
    se = jnp.sum(jnp.exp(lt - m), axis=0, keepdims=True)
    conf = 1.0 / se
    sidx = jax.lax.broadcasted_iota(jnp.int32, (c, _BN), 0)
    am = jnp.min(jnp.where(lt == m, sidx, c), axis=0, keepdims=True)
    correct = (am == yt_ref[0]).astype(jnp.float32)

    cf = conf.reshape(_R, 128)
    cr = correct.reshape(_R, 128)

    @pl.when(i < nb - 1)
    def _full():
        _binning(cf, cr, out_ref)

    @pl.when(i == nb - 1)
    def _tail():
        valid = n_rows - (nb - 1) * _BN
        rglob = (
            jax.lax.broadcasted_iota(jnp.int32, (_R, 128), 0) * 128
            + jax.lax.broadcasted_iota(jnp.int32, (_R, 128), 1)
        )
        _binning(jnp.where(rglob < valid, cf, 2.0), cr, out_ref)


def _combine_body(parts_ref, out_ref, *, inv_n: float):
    s = jnp.sum(parts_ref[...], axis=0)
    r = jnp.sum(s, axis=1, keepdims=True)
    cnt = r[0:_N_BINS, :] - r[1:_N_BINS + 1, :]
    dcf = r[16:16 + _N_BINS, :] - r[17:17 + _N_BINS, :]
    dac = r[32:32 + _N_BINS, :] - r[33:33 + _N_BINS, :]
    denom = jnp.maximum(cnt, 1.0)
    term = jnp.where(cnt > 0, jnp.abs(dcf - dac) / denom * (cnt * inv_n), 0.0)
    ece = jnp.sum(term, axis=0, keepdims=True)
    out_ref[...] = jnp.broadcast_to(ece, (1, 128))


def kernel(y_pred, y_true):
    n, c = y_pred.shape
    nb = (n + _BN - 1) // _BN

    yt = y_true.astype(jnp.int32)
    yt = jnp.pad(yt, (0, nb * _BN - n)).reshape(nb, 1, _BN)

    parts = pl.pallas_call(
        functools.partial(_pass1_body, n_rows=n, c=c),
        grid=(nb,),
        in_specs=[
            pl.BlockSpec((_BN, c), lambda i: (i, 0)),
            pl.BlockSpec((1, 1, _BN), lambda i: (i, 0, 0)),
        ],
        out_specs=pl.BlockSpec((1, 48, 128), lambda i: (i, 0, 0)),
        out_shape=jax.ShapeDtypeStruct((nb, 48, 128), jnp.float32),
        compiler_params=pltpu.CompilerParams(
            dimension_semantics=("parallel",),
        ),
    )(y_pred, yt)

    out = pl.pallas_call(
        functools.partial(_combine_body, inv_n=1.0 / n),
        grid=(1,),
        in_specs=[pl.BlockSpec((nb, 48, 128), lambda i: (0, 0, 0))],
        out_specs=pl.BlockSpec((1, 128), lambda i: (0, 0)),
        out_shape=jax.ShapeDtypeStruct((1, 128), jnp.float32),
    )(parts)
    return out[0, 0:1]


# speedup vs baseline: 1.7682x; 1.0287x over previous
import functools

import jax
import jax.numpy as jnp
import numpy as np
from jax.experimental import pallas as pl
from jax.experimental.pallas import tpu as pltpu

_N_BINS = 15
_BN = 16384
_R = _BN // 128
_BOUNDS = tuple(float(x) for x in np.linspace(0.0, 1.0, _N_BINS + 1, dtype=np.float32))


def _binning(cf, cr, out_ref):
    s_cnt, s_conf, s_acc = [], [], []
    one = jnp.ones_like(cf)
    zero = jnp.zeros_like(cf)
    for b in _BOUNDS:
        g = cf > b
        s_cnt.append(jnp.sum(jnp.where(g, one, zero), axis=0, keepdims=True))
        s_conf.append(jnp.sum(jnp.where(g, cf, zero), axis=0, keepdims=True))
        s_acc.append(jnp.sum(jnp.where(g, cr, zero), axis=0, keepdims=True))
    out_ref[0] = jnp.concatenate(s_cnt + s_conf + s_acc, axis=0)


def _pass1_body(yp_ref, yt_ref, out_ref, *, n_rows: int, c: int):
    i = pl.program_id(0)
    nb = pl.num_programs(0)

    logits = yp_ref[...]
    lt = jnp.transpose(logits, (1, 0))
    m = jnp.max(lt, axis=0, keepdims=True)
    se = jnp.sum(jnp.exp(lt - m), axis=0, keepdims=True)
    conf = 1.0 / se
    sidx = jax.lax.broadcasted_iota(jnp.int32, (c, _BN), 0)
    am = jnp.min(jnp.where(lt == m, sidx, c), axis=0, keepdims=True)
    correct = (am == yt_ref[0]).astype(jnp.float32)

    cf = conf.reshape(_R, 128)
    cr = correct.reshape(_R, 128)

    @pl.when(i < nb - 1)
    def _full():
        _binning(cf, cr, out_ref)

    @pl.when(i == nb - 1)
    def _tail():
        valid = n_rows - (nb - 1) * _BN
        rglob = (
            jax.lax.broadcasted_iota(jnp.int32, (_R, 128), 0) * 128
            + jax.lax.broadcasted_iota(jnp.int32, (_R, 128), 1)
        )
        _binning(jnp.where(rglob < valid, cf, 2.0), cr, out_ref)


def _combine_body(parts_ref, out_ref, *, inv_n: float):
    s = jnp.sum(parts_ref[...], axis=0)
    r = jnp.sum(s, axis=1, keepdims=True)
    cnt = r[0:_N_BINS, :] - r[1:_N_BINS + 1, :]
    dcf = r[16:16 + _N_BINS, :] - r[17:17 + _N_BINS, :]
    dac = r[32:32 + _N_BINS, :] - r[33:33 + _N_BINS, :]
    denom = jnp.maximum(cnt, 1.0)
    term = jnp.where(cnt > 0, jnp.abs(dcf - dac) / denom * (cnt * inv_n), 0.0)
    ece = jnp.sum(term, axis=0, keepdims=True)
    out_ref[...] = jnp.broadcast_to(ece, (1, 128))


def kernel(y_pred, y_true):
    n, c = y_pred.shape
    nb = (n + _BN - 1) // _BN

    yt = y_true.astype(jnp.int32)
    yt = jnp.pad(yt, (0, nb * _BN - n)).reshape(nb, 1, _BN)

    parts = pl.pallas_call(
        functools.partial(_pass1_body, n_rows=n, c=c),
        grid=(nb,),
        in_specs=[
            pl.BlockSpec((_BN, c), lambda i: (i, 0)),
            pl.BlockSpec((1, 1, _BN), lambda i: (i, 0, 0)),
        ],
        out_specs=pl.BlockSpec((1, 48, 128), lambda i: (i, 0, 0)),
        out_shape=jax.ShapeDtypeStruct((nb, 48, 128), jnp.float32),
        compiler_params=pltpu.CompilerParams(
            dimension_semantics=("parallel",),
        ),
    )(y_pred, yt)

    out = pl.pallas_call(
        functools.partial(_combine_body, inv_n=1.0 / n),
        grid=(1,),
        in_specs=[pl.BlockSpec((nb, 48, 128), lambda i: (0, 0, 0))],
        out_specs=pl.BlockSpec((1, 128), lambda i: (0, 0)),
        out_shape=jax.ShapeDtypeStruct((1, 128), jnp.float32),
    )(parts)
    return out[0, 0:1]
